# Initial kernel scaffold; baseline (speedup 1.0000x reference)
import functools

import numpy as np
import jax
import jax.numpy as jnp
from jax import lax
from jax.experimental import pallas as pl
from jax.experimental.pallas import tpu as pltpu
from jax.experimental.pallas import tpu_sc as plsc

_N = 10000
_E = 320000
_H = 128
_HEADS = 4
_C = 32

_NB = 1000
_EB = 1280

_NW = 32
_GCH = 80

_SC_JNP = False

_HI = jax.lax.Precision.HIGHEST

_SUMM = np.zeros((128, 4), np.float32)
for _h in range(4):
    _SUMM[32 * _h:32 * (_h + 1), _h] = 1.0
_BC = _SUMM.T.copy()
_EX16 = np.eye(4, 16, dtype=np.float32)
_DEN = np.zeros((16, 128), np.float32)
_DEN[:4] = _BC


def _full(a):
    nd = a.ndim
    return pl.BlockSpec(a.shape, lambda i, *_: (0,) * nd)


def _rows(block, ncols):
    return pl.BlockSpec((block, ncols), lambda i: (i, 0))


def _cparams():
    return pltpu.CompilerParams(dimension_semantics=("arbitrary",))


def _node_prep1_body(x_ref, w1, b1, w2, b2, wq, bq, wk, bk, wv, bv,
                     t1_ref, nf_ref):
    x = x_ref[...]
    h = jnp.maximum(jnp.dot(x, w1[...], preferred_element_type=jnp.float32)
                    + b1[...], 0.0)
    nf = jnp.maximum(jnp.dot(h, w2[...], preferred_element_type=jnp.float32)
                     + b2[...], 0.0)
    nf_ref[...] = nf
    t1_ref[:, 0:128] = jnp.dot(nf, wq[...], preferred_element_type=jnp.float32) + bq[...]
    t1_ref[:, 128:256] = jnp.dot(nf, wk[...], preferred_element_type=jnp.float32) + bk[...]
    t1_ref[:, 256:384] = jnp.dot(nf, wv[...], preferred_element_type=jnp.float32) + bv[...]


def _node_prep1(x, pn, ptc):
    (w1, b1), (w2, b2) = pn
    args = (x, w1, b1.reshape(1, -1), w2, b2.reshape(1, -1),
            ptc["q"][0], ptc["q"][1].reshape(1, -1),
            ptc["k"][0], ptc["k"][1].reshape(1, -1),
            ptc["v"][0], ptc["v"][1].reshape(1, -1))
    return pl.pallas_call(
        _node_prep1_body,
        grid=(_N // _NB,),
        in_specs=[_rows(_NB, 128)] + [_full(a) for a in args[1:]],
        out_specs=[_rows(_NB, 384), _rows(_NB, 128)],
        out_shape=[jax.ShapeDtypeStruct((_N, 384), jnp.float32),
                   jax.ShapeDtypeStruct((_N, 128), jnp.float32)],
        compiler_params=_cparams(),
    )(*args)


def _edge_emb_body(ea_ref, w1, b1, w2, b2, wp, bp, ef_ref, e1_ref):
    ea = ea_ref[...]
    h = jnp.maximum(jnp.dot(ea, w1[...], preferred_element_type=jnp.float32)
                    + b1[...], 0.0)
    ef = jnp.maximum(jnp.dot(h, w2[...], preferred_element_type=jnp.float32)
                     + b2[...], 0.0)
    ef_ref[...] = ef
    e1_ref[...] = jnp.dot(ef, wp[...], preferred_element_type=jnp.float32) + bp[...]


def _edge_emb(ea, pe, ptc):
    (w1, b1), (w2, b2) = pe
    args = (ea, w1, b1.reshape(1, -1), w2, b2.reshape(1, -1),
            ptc["e"][0], ptc["e"][1].reshape(1, -1))
    return pl.pallas_call(
        _edge_emb_body,
        grid=(_E // _EB,),
        in_specs=[_rows(_EB, 16)] + [_full(a) for a in args[1:]],
        out_specs=[_rows(_EB, 128), _rows(_EB, 128)],
        out_shape=[jax.ShapeDtypeStruct((_E, 128), jnp.float32),
                   jax.ShapeDtypeStruct((_E, 128), jnp.float32)],
        compiler_params=_cparams(),
    )(*args)


def _attn_body(gs_ref, gd_ref, ef_ref, eb_ref, summ, bc, ex16,
               mf_ref, xf_ref, mb_ref, xb_ref):
    isq = 1.0 / np.sqrt(np.float32(_C))
    gs = gs_ref[...]
    gd = gd_ref[...]

    def half(q, k, v, e, m_ref, x_ref):
        prod = q * (k + e)
        alpha = jnp.dot(prod, summ[...], precision=_HI,
                        preferred_element_type=jnp.float32) * isq
        ex = jnp.exp(alpha)
        exb = jnp.dot(ex, bc[...], precision=_HI,
                      preferred_element_type=jnp.float32)
        m_ref[...] = (v + e) * exb
        x_ref[...] = jnp.dot(ex, ex16[...], precision=_HI,
                             preferred_element_type=jnp.float32)

    half(gd[:, 0:128], gs[:, 128:256], gs[:, 256:384], ef_ref[...],
         mf_ref, xf_ref)
    half(gs[:, 0:128], gd[:, 128:256], gd[:, 256:384], eb_ref[...],
         mb_ref, xb_ref)


def _attn(gs, gd, e_f, e_b):
    args = (gs, gd, e_f, e_b, jnp.asarray(_SUMM), jnp.asarray(_BC),
            jnp.asarray(_EX16))
    return pl.pallas_call(
        _attn_body,
        grid=(_E // _EB,),
        in_specs=[_rows(_EB, 384), _rows(_EB, 384), _rows(_EB, 128),
                  _rows(_EB, 128)] + [_full(a) for a in args[4:]],
        out_specs=[_rows(_EB, 128), _rows(_EB, 16),
                   _rows(_EB, 128), _rows(_EB, 16)],
        out_shape=[jax.ShapeDtypeStruct((_E, 128), jnp.float32),
                   jax.ShapeDtypeStruct((_E, 16), jnp.float32),
                   jax.ShapeDtypeStruct((_E, 128), jnp.float32),
                   jax.ShapeDtypeStruct((_E, 16), jnp.float32)],
        compiler_params=_cparams(),
    )(*args)


def _finalize1_body(agg_ref, den_ref, nf_ref, ws, bs, g, b, den_m,
                    wq, bq, wk, bk, wv, bv, t2_ref, u_ref):
    agg = agg_ref[0] + agg_ref[1]
    den = den_ref[0] + den_ref[1]
    nf = nf_ref[...]
    denb = jnp.dot(den, den_m[...], precision=_HI,
                   preferred_element_type=jnp.float32)
    comb = agg / (denb + 1e-16) + jnp.dot(
        nf, ws[...], preferred_element_type=jnp.float32) + bs[...]
    mu = jnp.mean(comb, axis=-1, keepdims=True)
    cc = comb - mu
    var = jnp.mean(cc * cc, axis=-1, keepdims=True)
    comb = cc / jnp.sqrt(var + 1e-5) * g[...] + b[...]
    nf2 = comb + nf
    u_ref[:, 0:128] = comb
    u_ref[:, 128:256] = nf2
    t2_ref[:, 0:128] = jnp.dot(nf2, wq[...], preferred_element_type=jnp.float32) + bq[...]
    t2_ref[:, 128:256] = jnp.dot(nf2, wk[...], preferred_element_type=jnp.float32) + bk[...]
    t2_ref[:, 256:384] = jnp.dot(nf2, wv[...], preferred_element_type=jnp.float32) + bv[...]


def _finalize1(aggp, denp, nf, ptc1, ln, ptc2):
    args = (aggp, denp, nf,
            ptc1["skip"][0], ptc1["skip"][1].reshape(1, -1),
            ln[0].reshape(1, -1), ln[1].reshape(1, -1), jnp.asarray(_DEN),
            ptc2["q"][0], ptc2["q"][1].reshape(1, -1),
            ptc2["k"][0], ptc2["k"][1].reshape(1, -1),
            ptc2["v"][0], ptc2["v"][1].reshape(1, -1))
    return pl.pallas_call(
        _finalize1_body,
        grid=(_N // _NB,),
        in_specs=[pl.BlockSpec((2, _NB, 128), lambda i: (0, i, 0)),
                  pl.BlockSpec((2, _NB, 16), lambda i: (0, i, 0)),
                  _rows(_NB, 128)] + [_full(a) for a in args[3:]],
        out_specs=[_rows(_NB, 384), _rows(_NB, 256)],
        out_shape=[jax.ShapeDtypeStruct((_N, 384), jnp.float32),
                   jax.ShapeDtypeStruct((_N, 256), jnp.float32)],
        compiler_params=_cparams(),
    )(*args)


def _edge_stage_body(us_ref, ud_ref, ef0_ref,
                     wm1a, wm1b, bm1, wm2, bm2,
                     we1a, we1b, be1, we2, be2, wp, bp,
                     e2f_ref, e2b_ref, mom_ref):
    cs = us_ref[:, 0:128]
    cd = ud_ref[:, 0:128]
    ns = us_ref[:, 128:256]
    nd = ud_ref[:, 128:256]
    ef0 = ef0_ref[...]

    hm = jnp.maximum(jnp.dot(ns, wm1a[...], preferred_element_type=jnp.float32)
                     + jnp.dot(nd, wm1b[...], preferred_element_type=jnp.float32)
                     + bm1[...], 0.0)
    mom_ref[...] = jnp.dot(hm, wm2[...], preferred_element_type=jnp.float32) + bm2[...]

    def half(a, b, out_ref):
        h = jnp.maximum(jnp.dot(a, we1a[...], preferred_element_type=jnp.float32)
                        + jnp.dot(b, we1b[...], preferred_element_type=jnp.float32)
                        + be1[...], 0.0)
        ne = jnp.maximum(jnp.dot(h, we2[...], preferred_element_type=jnp.float32)
                         + be2[...], 0.0)
        ef2 = ne + ef0
        out_ref[...] = jnp.dot(ef2, wp[...], preferred_element_type=jnp.float32) + bp[...]

    half(cs, cd, e2f_ref)
    half(cd, cs, e2b_ref)


def _edge_stage(u_s, u_d, ef0, pmom, pedge, ptc2):
    (wm1, bm1), (wm2, bm2) = pmom
    (we1, be1), (we2, be2) = pedge
    wm2p = jnp.pad(wm2, ((0, 0), (0, 7)))
    bm2p = jnp.pad(bm2, (0, 7)).reshape(1, 8)
    args = (u_s, u_d, ef0,
            wm1[:128], wm1[128:], bm1.reshape(1, -1), wm2p, bm2p,
            we1[:128], we1[128:], be1.reshape(1, -1), we2,
            be2.reshape(1, -1),
            ptc2["e"][0], ptc2["e"][1].reshape(1, -1))
    return pl.pallas_call(
        _edge_stage_body,
        grid=(_E // _EB,),
        in_specs=[_rows(_EB, 256), _rows(_EB, 256), _rows(_EB, 128)]
        + [_full(a) for a in args[3:]],
        out_specs=[_rows(_EB, 128), _rows(_EB, 128), _rows(_EB, 8)],
        out_shape=[jax.ShapeDtypeStruct((_E, 128), jnp.float32),
                   jax.ShapeDtypeStruct((_E, 128), jnp.float32),
                   jax.ShapeDtypeStruct((_E, 8), jnp.float32)],
        compiler_params=_cparams(),
    )(*args)


def _finalize2_body(agg_ref, den_ref, u_ref, ws, bs, den_m, fin_ref):
    agg = agg_ref[0] + agg_ref[1]
    den = den_ref[0] + den_ref[1]
    nf2 = u_ref[:, 128:256]
    denb = jnp.dot(den, den_m[...], precision=_HI,
                   preferred_element_type=jnp.float32)
    fin_ref[...] = agg / (denb + 1e-16) + jnp.dot(
        nf2, ws[...], preferred_element_type=jnp.float32) + bs[...]


def _finalize2(aggp, denp, u, ptc2):
    args = (aggp, denp, u, ptc2["skip"][0], ptc2["skip"][1].reshape(1, -1),
            jnp.asarray(_DEN))
    return pl.pallas_call(
        _finalize2_body,
        grid=(_N // _NB,),
        in_specs=[pl.BlockSpec((2, _NB, 128), lambda i: (0, i, 0)),
                  pl.BlockSpec((2, _NB, 16), lambda i: (0, i, 0)),
                  _rows(_NB, 256)] + [_full(a) for a in args[3:]],
        out_specs=[_rows(_NB, 128)],
        out_shape=[jax.ShapeDtypeStruct((_N, 128), jnp.float32)],
        compiler_params=_cparams(),
    )(*args)[0]


def _head_body(fs_ref, fd_ref, w1a, w1b, b1, w2, b2, out_ref):
    h = jnp.maximum(jnp.dot(fs_ref[...], w1a[...], preferred_element_type=jnp.float32)
                    + jnp.dot(fd_ref[...], w1b[...], preferred_element_type=jnp.float32)
                    + b1[...], 0.0)
    out_ref[...] = jnp.dot(h, w2[...], preferred_element_type=jnp.float32) + b2[...]


def _head(f_s, f_d, p):
    (w1, b1), (w2, b2) = p
    w2p = jnp.pad(w2, ((0, 0), (0, 7)))
    b2p = jnp.pad(b2, (0, 7)).reshape(1, 8)
    args = (f_s, f_d, w1[:128], w1[128:], b1.reshape(1, -1), w2p, b2p)
    return pl.pallas_call(
        _head_body,
        grid=(_E // _EB,),
        in_specs=[_rows(_EB, 128), _rows(_EB, 128)]
        + [_full(a) for a in args[2:]],
        out_specs=[_rows(_EB, 8)],
        out_shape=[jax.ShapeDtypeStruct((_E, 8), jnp.float32)],
        compiler_params=_cparams(),
    )(*args)[0]


def _sc_gather(table, idx):
    if _SC_JNP:
        return jnp.take(table, idx, axis=0)
    m = idx.shape[0]
    d = table.shape[1]
    per_w = m // _NW
    nch = per_w // _GCH

    @functools.partial(
        pl.kernel,
        out_type=jax.ShapeDtypeStruct((m, d), jnp.float32),
        mesh=plsc.VectorSubcoreMesh(core_axis_name="c", subcore_axis_name="s"),
        scratch_types=[pltpu.VMEM((_GCH,), jnp.int32),
                       pltpu.VMEM((_GCH, d), jnp.float32),
                       pltpu.SemaphoreType.DMA],
    )
    def k(table_hbm, idx_hbm, out_hbm, idx_v, rows_v, sem):
        wid = lax.axis_index("s") * 2 + lax.axis_index("c")
        base = wid * per_w

        @pl.loop(0, nch)
        def _(i):
            b = base + i * _GCH
            pltpu.sync_copy(idx_hbm.at[pl.ds(b, _GCH)], idx_v)
            pltpu.async_copy(table_hbm.at[idx_v], rows_v, sem).wait()
            pltpu.sync_copy(rows_v, out_hbm.at[pl.ds(b, _GCH)])

    return k(table, idx)


def _sc_scatter_add(msg_f, ex_f, idx_f, msg_b, ex_b, idx_b):
    if _SC_JNP:
        seg = jnp.zeros((_N, 128), jnp.float32).at[idx_f].add(msg_f).at[idx_b].add(msg_b)
        den = jnp.zeros((_N, 16), jnp.float32).at[idx_f].add(ex_f).at[idx_b].add(ex_b)
        return (jnp.stack([seg, jnp.zeros_like(seg)]),
                jnp.stack([den, jnp.zeros_like(den)]))
    m = msg_f.shape[0]
    half = m // 2
    per_s = half // 16
    nch = per_s // _GCH
    zr = _N // 16
    zeros = jnp.zeros((_N, 128), jnp.float32)
    zeros16 = jnp.zeros((_N, 16), jnp.float32)

    @functools.partial(
        pl.kernel,
        out_type=[jax.ShapeDtypeStruct((2, _N, 128), jnp.float32),
                  jax.ShapeDtypeStruct((2, _N, 16), jnp.float32)],
        mesh=plsc.VectorSubcoreMesh(core_axis_name="c", subcore_axis_name="s"),
        scratch_types=[pltpu.VMEM((_GCH,), jnp.int32),
                       pltpu.VMEM((_GCH, 128), jnp.float32),
                       pltpu.VMEM((_GCH, 16), jnp.float32),
                       pltpu.VMEM_SHARED((_N, 128), jnp.float32),
                       pltpu.VMEM_SHARED((_N, 16), jnp.float32),
                       pltpu.SemaphoreType.DMA],
    )
    def k(mf_hbm, xf_hbm, if_hbm, mb_hbm, xb_hbm, ib_hbm, z_hbm, z16_hbm,
          agg_hbm, den_hbm, idx_v, rows_v, ex_v, acc_sh, den_sh, sem):
        c = lax.axis_index("c")
        s = lax.axis_index("s")
        pltpu.sync_copy(z_hbm.at[pl.ds(s * zr, zr)],
                        acc_sh.at[pl.ds(s * zr, zr)])
        pltpu.sync_copy(z16_hbm.at[pl.ds(s * zr, zr)],
                        den_sh.at[pl.ds(s * zr, zr)])
        plsc.subcore_barrier()
        base = c * half + s * per_s
        for msg_hbm, x_hbm, i_hbm in ((mf_hbm, xf_hbm, if_hbm),
                                      (mb_hbm, xb_hbm, ib_hbm)):
            @pl.loop(0, nch)
            def _(i, msg_hbm=msg_hbm, x_hbm=x_hbm, i_hbm=i_hbm):
                b = base + i * _GCH
                pltpu.sync_copy(i_hbm.at[pl.ds(b, _GCH)], idx_v)
                pltpu.sync_copy(msg_hbm.at[pl.ds(b, _GCH)], rows_v)
                pltpu.sync_copy(x_hbm.at[pl.ds(b, _GCH)], ex_v)
                pltpu.sync_copy(rows_v, acc_sh.at[idx_v], add=True)
                pltpu.sync_copy(ex_v, den_sh.at[idx_v], add=True)
        plsc.subcore_barrier()
        pltpu.sync_copy(acc_sh.at[pl.ds(s * zr, zr)],
                        agg_hbm.at[c].at[pl.ds(s * zr, zr)])
        pltpu.sync_copy(den_sh.at[pl.ds(s * zr, zr)],
                        den_hbm.at[c].at[pl.ds(s * zr, zr)])

    return k(msg_f, ex_f, idx_f, msg_b, ex_b, idx_b, zeros, zeros16)


def kernel(x, edge_index, edge_attr, params):
    src0 = edge_index[0]
    dst0 = edge_index[1]

    t1, nf = _node_prep1(x, params["node_emb"], params["mom_tc"])
    ef0, e1 = _edge_emb(edge_attr, params["edge_emb"], params["mom_tc"])

    g_s = _sc_gather(t1, src0)
    g_d = _sc_gather(t1, dst0)
    m_f, x_f, m_b, x_b = _attn(g_s, g_d, e1, e1)
    aggp, denp = _sc_scatter_add(m_f, x_f, dst0, m_b, x_b, src0)
    t2, u = _finalize1(aggp, denp, nf, params["mom_tc"], params["mom_ln"],
                       params["edge_tc"])

    u_s = _sc_gather(u, src0)
    u_d = _sc_gather(u, dst0)
    e2_f, e2_b, mom8 = _edge_stage(u_s, u_d, ef0, params["mom_reg"],
                                   params["edge_mlp"], params["edge_tc"])

    t_s = _sc_gather(t2, src0)
    t_d = _sc_gather(t2, dst0)
    m_f2, x_f2, m_b2, x_b2 = _attn(t_s, t_d, e2_f, e2_b)
    aggp2, denp2 = _sc_scatter_add(m_f2, x_f2, dst0, m_b2, x_b2, src0)
    fin = _finalize2(aggp2, denp2, u, params["edge_tc"])

    f_s = _sc_gather(fin, src0)
    f_d = _sc_gather(fin, dst0)
    sc8 = _head(f_s, f_d, params["edge_cls"])

    return (mom8[:, 0], sc8[:, 0])


# speedup vs baseline: 24.1525x; 24.1525x over previous
import functools

import numpy as np
import jax
import jax.numpy as jnp
from jax import lax
from jax.experimental import pallas as pl
from jax.experimental.pallas import tpu as pltpu
from jax.experimental.pallas import tpu_sc as plsc

_N = 10000
_E = 320000
_H = 128
_HEADS = 4
_C = 32

_NB = 1000
_EB = 1280

_NP = 10240

_NW = 32
_GCH = 80

_SC_JNP = False
_SCATTER_JNP = False

_HI = jax.lax.Precision.HIGHEST

_SUMM = np.zeros((128, 4), np.float32)
for _h in range(4):
    _SUMM[32 * _h:32 * (_h + 1), _h] = 1.0
_BC = _SUMM.T.copy()
_EX128 = np.eye(4, 128, dtype=np.float32)
_DEN = np.zeros((16, 128), np.float32)
_DEN[:4] = _BC


def _full(a):
    nd = a.ndim
    return pl.BlockSpec(a.shape, lambda i, *_: (0,) * nd)


def _rows(block, ncols):
    return pl.BlockSpec((block, ncols), lambda i: (i, 0))


def _cparams():
    return pltpu.CompilerParams(dimension_semantics=("arbitrary",))


def _node_prep1_body(x_ref, w1, b1, w2, b2, wq, bq, wk, bk, wv, bv,
                     t1_ref, nf_ref):
    x = x_ref[...]
    h = jnp.maximum(jnp.dot(x, w1[...], preferred_element_type=jnp.float32)
                    + b1[...], 0.0)
    nf = jnp.maximum(jnp.dot(h, w2[...], preferred_element_type=jnp.float32)
                     + b2[...], 0.0)
    nf_ref[...] = nf
    t1_ref[:, 0:128] = jnp.dot(nf, wq[...], preferred_element_type=jnp.float32) + bq[...]
    t1_ref[:, 128:256] = jnp.dot(nf, wk[...], preferred_element_type=jnp.float32) + bk[...]
    t1_ref[:, 256:384] = jnp.dot(nf, wv[...], preferred_element_type=jnp.float32) + bv[...]


def _node_prep1(x, pn, ptc):
    (w1, b1), (w2, b2) = pn
    args = (x, w1, b1.reshape(1, -1), w2, b2.reshape(1, -1),
            ptc["q"][0], ptc["q"][1].reshape(1, -1),
            ptc["k"][0], ptc["k"][1].reshape(1, -1),
            ptc["v"][0], ptc["v"][1].reshape(1, -1))
    return pl.pallas_call(
        _node_prep1_body,
        grid=(_N // _NB,),
        in_specs=[_rows(_NB, 128)] + [_full(a) for a in args[1:]],
        out_specs=[_rows(_NB, 384), _rows(_NB, 128)],
        out_shape=[jax.ShapeDtypeStruct((_N, 384), jnp.float32),
                   jax.ShapeDtypeStruct((_N, 128), jnp.float32)],
        compiler_params=_cparams(),
    )(*args)


def _edge_emb_body(ea_ref, w1, b1, w2, b2, wp, bp, ef_ref, e1_ref):
    ea = ea_ref[...]
    h = jnp.maximum(jnp.dot(ea, w1[...], preferred_element_type=jnp.float32)
                    + b1[...], 0.0)
    ef = jnp.maximum(jnp.dot(h, w2[...], preferred_element_type=jnp.float32)
                     + b2[...], 0.0)
    ef_ref[...] = ef
    e1_ref[...] = jnp.dot(ef, wp[...], preferred_element_type=jnp.float32) + bp[...]


def _edge_emb(ea, pe, ptc):
    (w1, b1), (w2, b2) = pe
    args = (ea, w1, b1.reshape(1, -1), w2, b2.reshape(1, -1),
            ptc["e"][0], ptc["e"][1].reshape(1, -1))
    return pl.pallas_call(
        _edge_emb_body,
        grid=(_E // _EB,),
        in_specs=[_rows(_EB, 16)] + [_full(a) for a in args[1:]],
        out_specs=[_rows(_EB, 128), _rows(_EB, 128)],
        out_shape=[jax.ShapeDtypeStruct((_E, 128), jnp.float32),
                   jax.ShapeDtypeStruct((_E, 128), jnp.float32)],
        compiler_params=_cparams(),
    )(*args)


def _attn_body(gs_ref, gd_ref, ef_ref, eb_ref, summ, bc, ex128,
               mf_ref, xf_ref, mb_ref, xb_ref):
    isq = 1.0 / np.sqrt(np.float32(_C))
    gs = gs_ref[...]
    gd = gd_ref[...]

    def half(q, k, v, e, m_ref, x_ref):
        prod = q * (k + e)
        alpha = jnp.dot(prod, summ[...], precision=_HI,
                        preferred_element_type=jnp.float32) * isq
        ex = jnp.exp(alpha)
        exb = jnp.dot(ex, bc[...], precision=_HI,
                      preferred_element_type=jnp.float32)
        m_ref[...] = (v + e) * exb
        x_ref[...] = jnp.dot(ex, ex128[...], precision=_HI,
                             preferred_element_type=jnp.float32)

    half(gd[:, 0:128], gs[:, 128:256], gs[:, 256:384], ef_ref[...],
         mf_ref, xf_ref)
    half(gs[:, 0:128], gd[:, 128:256], gd[:, 256:384], eb_ref[...],
         mb_ref, xb_ref)


def _attn(gs, gd, e_f, e_b):
    args = (gs, gd, e_f, e_b, jnp.asarray(_SUMM), jnp.asarray(_BC),
            jnp.asarray(_EX128))
    return pl.pallas_call(
        _attn_body,
        grid=(_E // _EB,),
        in_specs=[_rows(_EB, 384), _rows(_EB, 384), _rows(_EB, 128),
                  _rows(_EB, 128)] + [_full(a) for a in args[4:]],
        out_specs=[_rows(_EB, 128), _rows(_EB, 128),
                   _rows(_EB, 128), _rows(_EB, 128)],
        out_shape=[jax.ShapeDtypeStruct((_E, 128), jnp.float32),
                   jax.ShapeDtypeStruct((_E, 128), jnp.float32),
                   jax.ShapeDtypeStruct((_E, 128), jnp.float32),
                   jax.ShapeDtypeStruct((_E, 128), jnp.float32)],
        compiler_params=_cparams(),
    )(*args)


def _finalize1_body(agg_ref, den_ref, nf_ref, ws, bs, g, b, den_m,
                    wq, bq, wk, bk, wv, bv, t2_ref, u_ref):
    agg = agg_ref[0] + agg_ref[1]
    den = den_ref[0, :, 0:16] + den_ref[1, :, 0:16]
    nf = nf_ref[...]
    denb = jnp.dot(den, den_m[...], precision=_HI,
                   preferred_element_type=jnp.float32)
    comb = agg / (denb + 1e-16) + jnp.dot(
        nf, ws[...], preferred_element_type=jnp.float32) + bs[...]
    mu = jnp.mean(comb, axis=-1, keepdims=True)
    cc = comb - mu
    var = jnp.mean(cc * cc, axis=-1, keepdims=True)
    comb = cc / jnp.sqrt(var + 1e-5) * g[...] + b[...]
    nf2 = comb + nf
    u_ref[:, 0:128] = comb
    u_ref[:, 128:256] = nf2
    t2_ref[:, 0:128] = jnp.dot(nf2, wq[...], preferred_element_type=jnp.float32) + bq[...]
    t2_ref[:, 128:256] = jnp.dot(nf2, wk[...], preferred_element_type=jnp.float32) + bk[...]
    t2_ref[:, 256:384] = jnp.dot(nf2, wv[...], preferred_element_type=jnp.float32) + bv[...]


def _finalize1(aggp, denp, nf, ptc1, ln, ptc2):
    args = (aggp, denp, nf,
            ptc1["skip"][0], ptc1["skip"][1].reshape(1, -1),
            ln[0].reshape(1, -1), ln[1].reshape(1, -1), jnp.asarray(_DEN),
            ptc2["q"][0], ptc2["q"][1].reshape(1, -1),
            ptc2["k"][0], ptc2["k"][1].reshape(1, -1),
            ptc2["v"][0], ptc2["v"][1].reshape(1, -1))
    return pl.pallas_call(
        _finalize1_body,
        grid=(_N // _NB,),
        in_specs=[pl.BlockSpec((2, _NB, 128), lambda i: (0, i, 0)),
                  pl.BlockSpec((2, _NB, 128), lambda i: (0, i, 0)),
                  _rows(_NB, 128)] + [_full(a) for a in args[3:]],
        out_specs=[_rows(_NB, 384), _rows(_NB, 256)],
        out_shape=[jax.ShapeDtypeStruct((_N, 384), jnp.float32),
                   jax.ShapeDtypeStruct((_N, 256), jnp.float32)],
        compiler_params=_cparams(),
    )(*args)


def _edge_stage_body(us_ref, ud_ref, ef0_ref,
                     wm1a, wm1b, bm1, wm2, bm2,
                     we1a, we1b, be1, we2, be2, wp, bp,
                     e2f_ref, e2b_ref, mom_ref):
    cs = us_ref[:, 0:128]
    cd = ud_ref[:, 0:128]
    ns = us_ref[:, 128:256]
    nd = ud_ref[:, 128:256]
    ef0 = ef0_ref[...]

    hm = jnp.maximum(jnp.dot(ns, wm1a[...], preferred_element_type=jnp.float32)
                     + jnp.dot(nd, wm1b[...], preferred_element_type=jnp.float32)
                     + bm1[...], 0.0)
    mom_ref[...] = jnp.dot(hm, wm2[...], preferred_element_type=jnp.float32) + bm2[...]

    def half(a, b, out_ref):
        h = jnp.maximum(jnp.dot(a, we1a[...], preferred_element_type=jnp.float32)
                        + jnp.dot(b, we1b[...], preferred_element_type=jnp.float32)
                        + be1[...], 0.0)
        ne = jnp.maximum(jnp.dot(h, we2[...], preferred_element_type=jnp.float32)
                         + be2[...], 0.0)
        ef2 = ne + ef0
        out_ref[...] = jnp.dot(ef2, wp[...], preferred_element_type=jnp.float32) + bp[...]

    half(cs, cd, e2f_ref)
    half(cd, cs, e2b_ref)


def _edge_stage(u_s, u_d, ef0, pmom, pedge, ptc2):
    (wm1, bm1), (wm2, bm2) = pmom
    (we1, be1), (we2, be2) = pedge
    wm2p = jnp.pad(wm2, ((0, 0), (0, 7)))
    bm2p = jnp.pad(bm2, (0, 7)).reshape(1, 8)
    args = (u_s, u_d, ef0,
            wm1[:128], wm1[128:], bm1.reshape(1, -1), wm2p, bm2p,
            we1[:128], we1[128:], be1.reshape(1, -1), we2,
            be2.reshape(1, -1),
            ptc2["e"][0], ptc2["e"][1].reshape(1, -1))
    return pl.pallas_call(
        _edge_stage_body,
        grid=(_E // _EB,),
        in_specs=[_rows(_EB, 256), _rows(_EB, 256), _rows(_EB, 128)]
        + [_full(a) for a in args[3:]],
        out_specs=[_rows(_EB, 128), _rows(_EB, 128), _rows(_EB, 8)],
        out_shape=[jax.ShapeDtypeStruct((_E, 128), jnp.float32),
                   jax.ShapeDtypeStruct((_E, 128), jnp.float32),
                   jax.ShapeDtypeStruct((_E, 8), jnp.float32)],
        compiler_params=_cparams(),
    )(*args)


def _finalize2_body(agg_ref, den_ref, u_ref, ws, bs, den_m, fin_ref):
    agg = agg_ref[0] + agg_ref[1]
    den = den_ref[0, :, 0:16] + den_ref[1, :, 0:16]
    nf2 = u_ref[:, 128:256]
    denb = jnp.dot(den, den_m[...], precision=_HI,
                   preferred_element_type=jnp.float32)
    fin_ref[...] = agg / (denb + 1e-16) + jnp.dot(
        nf2, ws[...], preferred_element_type=jnp.float32) + bs[...]


def _finalize2(aggp, denp, u, ptc2):
    args = (aggp, denp, u, ptc2["skip"][0], ptc2["skip"][1].reshape(1, -1),
            jnp.asarray(_DEN))
    return pl.pallas_call(
        _finalize2_body,
        grid=(_N // _NB,),
        in_specs=[pl.BlockSpec((2, _NB, 128), lambda i: (0, i, 0)),
                  pl.BlockSpec((2, _NB, 128), lambda i: (0, i, 0)),
                  _rows(_NB, 256)] + [_full(a) for a in args[3:]],
        out_specs=[_rows(_NB, 128)],
        out_shape=[jax.ShapeDtypeStruct((_N, 128), jnp.float32)],
        compiler_params=_cparams(),
    )(*args)[0]


def _head_body(fs_ref, fd_ref, w1a, w1b, b1, w2, b2, out_ref):
    h = jnp.maximum(jnp.dot(fs_ref[...], w1a[...], preferred_element_type=jnp.float32)
                    + jnp.dot(fd_ref[...], w1b[...], preferred_element_type=jnp.float32)
                    + b1[...], 0.0)
    out_ref[...] = jnp.dot(h, w2[...], preferred_element_type=jnp.float32) + b2[...]


def _head(f_s, f_d, p):
    (w1, b1), (w2, b2) = p
    w2p = jnp.pad(w2, ((0, 0), (0, 7)))
    b2p = jnp.pad(b2, (0, 7)).reshape(1, 8)
    args = (f_s, f_d, w1[:128], w1[128:], b1.reshape(1, -1), w2p, b2p)
    return pl.pallas_call(
        _head_body,
        grid=(_E // _EB,),
        in_specs=[_rows(_EB, 128), _rows(_EB, 128)]
        + [_full(a) for a in args[2:]],
        out_specs=[_rows(_EB, 8)],
        out_shape=[jax.ShapeDtypeStruct((_E, 8), jnp.float32)],
        compiler_params=_cparams(),
    )(*args)[0]


def _sc_gather(table, idx):
    if _SC_JNP:
        return jnp.take(table, idx, axis=0)
    m = idx.shape[0]
    d = table.shape[1]
    per_w = m // _NW
    nch = per_w // _GCH

    @functools.partial(
        pl.kernel,
        out_type=jax.ShapeDtypeStruct((m, d), jnp.float32),
        mesh=plsc.VectorSubcoreMesh(core_axis_name="c", subcore_axis_name="s"),
        scratch_types=[pltpu.VMEM((_GCH,), jnp.int32),
                       pltpu.VMEM((_GCH, d), jnp.float32),
                       pltpu.SemaphoreType.DMA],
    )
    def k(table_hbm, idx_hbm, out_hbm, idx_v, rows_v, sem):
        wid = lax.axis_index("s") * 2 + lax.axis_index("c")
        base = wid * per_w

        @pl.loop(0, nch)
        def _(i):
            b = base + i * _GCH
            pltpu.sync_copy(idx_hbm.at[pl.ds(b, _GCH)], idx_v)
            pltpu.async_copy(table_hbm.at[idx_v], rows_v, sem).wait()
            pltpu.sync_copy(rows_v, out_hbm.at[pl.ds(b, _GCH)])

    return k(table, idx)


def _sc_scatter_add(msg_f, idx_f, msg_b, idx_b):
    if _SC_JNP or _SCATTER_JNP:
        seg = jnp.zeros((_NP, 128), jnp.float32).at[idx_f].add(msg_f).at[idx_b].add(msg_b)
        return jnp.stack([seg, jnp.zeros_like(seg)])
    m = msg_f.shape[0]
    half = m // 2
    per_s = half // 16
    nch = per_s // _GCH
    zr = _NP // 16
    nzch = zr // _GCH
    zeros = jnp.zeros((_GCH, 128), jnp.float32)

    @functools.partial(
        pl.kernel,
        out_type=jax.ShapeDtypeStruct((2, _NP, 128), jnp.float32),
        mesh=plsc.VectorSubcoreMesh(core_axis_name="c", subcore_axis_name="s"),
        scratch_types=[pltpu.VMEM((_GCH,), jnp.int32),
                       pltpu.VMEM((_GCH, 128), jnp.float32),
                       pltpu.VMEM_SHARED((_NP, 128), jnp.float32),
                       pltpu.SemaphoreType.DMA],
    )
    def k(mf_hbm, if_hbm, mb_hbm, ib_hbm, z_hbm,
          agg_hbm, idx_v, rows_v, acc_sh, sem):
        c = lax.axis_index("c")
        s = lax.axis_index("s")
        pltpu.sync_copy(z_hbm, rows_v)

        @pl.loop(0, nzch)
        def _(j):
            pltpu.sync_copy(rows_v, acc_sh.at[pl.ds(s * zr + j * _GCH, _GCH)])

        plsc.subcore_barrier()
        base = c * half + s * per_s
        for msg_hbm, i_hbm in ((mf_hbm, if_hbm), (mb_hbm, ib_hbm)):
            @pl.loop(0, nch)
            def _(i, msg_hbm=msg_hbm, i_hbm=i_hbm):
                b = base + i * _GCH
                pltpu.sync_copy(i_hbm.at[pl.ds(b, _GCH)], idx_v)
                pltpu.sync_copy(msg_hbm.at[pl.ds(b, _GCH)], rows_v)
                pltpu.sync_copy(rows_v, acc_sh.at[idx_v], add=True)
        plsc.subcore_barrier()

        @pl.loop(0, nzch)
        def _(j):
            o = s * zr + j * _GCH
            pltpu.sync_copy(acc_sh.at[pl.ds(o, _GCH)], rows_v)
            pltpu.sync_copy(rows_v, agg_hbm.at[c, pl.ds(o, _GCH)])

    return k(msg_f, idx_f, msg_b, idx_b, zeros)


def kernel(x, edge_index, edge_attr, params):
    src0 = edge_index[0]
    dst0 = edge_index[1]

    t1, nf = _node_prep1(x, params["node_emb"], params["mom_tc"])
    ef0, e1 = _edge_emb(edge_attr, params["edge_emb"], params["mom_tc"])

    g_s = _sc_gather(t1, src0)
    g_d = _sc_gather(t1, dst0)
    m_f, x_f, m_b, x_b = _attn(g_s, g_d, e1, e1)
    aggp = _sc_scatter_add(m_f, dst0, m_b, src0)
    denp = _sc_scatter_add(x_f, dst0, x_b, src0)
    t2, u = _finalize1(aggp, denp, nf, params["mom_tc"], params["mom_ln"],
                       params["edge_tc"])

    u_s = _sc_gather(u, src0)
    u_d = _sc_gather(u, dst0)
    e2_f, e2_b, mom8 = _edge_stage(u_s, u_d, ef0, params["mom_reg"],
                                   params["edge_mlp"], params["edge_tc"])

    t_s = _sc_gather(t2, src0)
    t_d = _sc_gather(t2, dst0)
    m_f2, x_f2, m_b2, x_b2 = _attn(t_s, t_d, e2_f, e2_b)
    aggp2 = _sc_scatter_add(m_f2, dst0, m_b2, src0)
    denp2 = _sc_scatter_add(x_f2, dst0, x_b2, src0)
    fin = _finalize2(aggp2, denp2, u, params["edge_tc"])

    f_s = _sc_gather(fin, src0)
    f_d = _sc_gather(fin, dst0)
    sc8 = _head(f_s, f_d, params["edge_cls"])

    return (mom8[:, 0], sc8[:, 0])


# speedup vs baseline: 24.1708x; 1.0008x over previous
import functools

import numpy as np
import jax
import jax.numpy as jnp
from jax import lax
from jax.experimental import pallas as pl
from jax.experimental.pallas import tpu as pltpu
from jax.experimental.pallas import tpu_sc as plsc

_N = 10000
_E = 320000
_H = 128
_HEADS = 4
_C = 32

_NB = 1000
_EB = 1280

_NP = 10240

_NW = 32
_GCH = 40

_SC_JNP = False
_SCATTER_JNP = False

_HI = jax.lax.Precision.HIGHEST

_SUMM = np.zeros((128, 4), np.float32)
for _h in range(4):
    _SUMM[32 * _h:32 * (_h + 1), _h] = 1.0
_BC = _SUMM.T.copy()
_EX128 = np.eye(4, 128, dtype=np.float32)
_DEN = np.zeros((16, 128), np.float32)
_DEN[:4] = _BC


def _full(a):
    nd = a.ndim
    return pl.BlockSpec(a.shape, lambda i, *_: (0,) * nd)


def _rows(block, ncols):
    return pl.BlockSpec((block, ncols), lambda i: (i, 0))


def _cparams():
    return pltpu.CompilerParams(dimension_semantics=("arbitrary",))


def _node_prep1_body(x_ref, w1, b1, w2, b2, wq, bq, wk, bk, wv, bv,
                     t1_ref, nf_ref):
    x = x_ref[...]
    h = jnp.maximum(jnp.dot(x, w1[...], preferred_element_type=jnp.float32)
                    + b1[...], 0.0)
    nf = jnp.maximum(jnp.dot(h, w2[...], preferred_element_type=jnp.float32)
                     + b2[...], 0.0)
    nf_ref[...] = nf
    t1_ref[:, 0:128] = jnp.dot(nf, wq[...], preferred_element_type=jnp.float32) + bq[...]
    t1_ref[:, 128:256] = jnp.dot(nf, wk[...], preferred_element_type=jnp.float32) + bk[...]
    t1_ref[:, 256:384] = jnp.dot(nf, wv[...], preferred_element_type=jnp.float32) + bv[...]


def _node_prep1(x, pn, ptc):
    (w1, b1), (w2, b2) = pn
    args = (x, w1, b1.reshape(1, -1), w2, b2.reshape(1, -1),
            ptc["q"][0], ptc["q"][1].reshape(1, -1),
            ptc["k"][0], ptc["k"][1].reshape(1, -1),
            ptc["v"][0], ptc["v"][1].reshape(1, -1))
    return pl.pallas_call(
        _node_prep1_body,
        grid=(_N // _NB,),
        in_specs=[_rows(_NB, 128)] + [_full(a) for a in args[1:]],
        out_specs=[_rows(_NB, 384), _rows(_NB, 128)],
        out_shape=[jax.ShapeDtypeStruct((_N, 384), jnp.float32),
                   jax.ShapeDtypeStruct((_N, 128), jnp.float32)],
        compiler_params=_cparams(),
    )(*args)


def _edge_emb_body(ea_ref, w1, b1, w2, b2, wp, bp, ef_ref, e1_ref):
    ea = ea_ref[...]
    h = jnp.maximum(jnp.dot(ea, w1[...], preferred_element_type=jnp.float32)
                    + b1[...], 0.0)
    ef = jnp.maximum(jnp.dot(h, w2[...], preferred_element_type=jnp.float32)
                     + b2[...], 0.0)
    ef_ref[...] = ef
    e1_ref[...] = jnp.dot(ef, wp[...], preferred_element_type=jnp.float32) + bp[...]


def _edge_emb(ea, pe, ptc):
    (w1, b1), (w2, b2) = pe
    args = (ea, w1, b1.reshape(1, -1), w2, b2.reshape(1, -1),
            ptc["e"][0], ptc["e"][1].reshape(1, -1))
    return pl.pallas_call(
        _edge_emb_body,
        grid=(_E // _EB,),
        in_specs=[_rows(_EB, 16)] + [_full(a) for a in args[1:]],
        out_specs=[_rows(_EB, 128), _rows(_EB, 128)],
        out_shape=[jax.ShapeDtypeStruct((_E, 128), jnp.float32),
                   jax.ShapeDtypeStruct((_E, 128), jnp.float32)],
        compiler_params=_cparams(),
    )(*args)


def _attn_body(gs_ref, gd_ref, ef_ref, eb_ref, summ, bc, ex128,
               mf_ref, xf_ref, mb_ref, xb_ref):
    isq = 1.0 / np.sqrt(np.float32(_C))
    gs = gs_ref[...]
    gd = gd_ref[...]

    def half(q, k, v, e, m_ref, x_ref):
        prod = q * (k + e)
        alpha = jnp.dot(prod, summ[...], precision=_HI,
                        preferred_element_type=jnp.float32) * isq
        ex = jnp.exp(alpha)
        exb = jnp.dot(ex, bc[...], precision=_HI,
                      preferred_element_type=jnp.float32)
        m_ref[...] = (v + e) * exb
        x_ref[...] = jnp.dot(ex, ex128[...], precision=_HI,
                             preferred_element_type=jnp.float32)

    half(gd[:, 0:128], gs[:, 128:256], gs[:, 256:384], ef_ref[...],
         mf_ref, xf_ref)
    half(gs[:, 0:128], gd[:, 128:256], gd[:, 256:384], eb_ref[...],
         mb_ref, xb_ref)


def _attn(gs, gd, e_f, e_b):
    args = (gs, gd, e_f, e_b, jnp.asarray(_SUMM), jnp.asarray(_BC),
            jnp.asarray(_EX128))
    return pl.pallas_call(
        _attn_body,
        grid=(_E // _EB,),
        in_specs=[_rows(_EB, 384), _rows(_EB, 384), _rows(_EB, 128),
                  _rows(_EB, 128)] + [_full(a) for a in args[4:]],
        out_specs=[_rows(_EB, 128), _rows(_EB, 128),
                   _rows(_EB, 128), _rows(_EB, 128)],
        out_shape=[jax.ShapeDtypeStruct((_E, 128), jnp.float32),
                   jax.ShapeDtypeStruct((_E, 128), jnp.float32),
                   jax.ShapeDtypeStruct((_E, 128), jnp.float32),
                   jax.ShapeDtypeStruct((_E, 128), jnp.float32)],
        compiler_params=_cparams(),
    )(*args)


def _finalize1_body(agg_ref, den_ref, nf_ref, ws, bs, g, b, den_m,
                    wq, bq, wk, bk, wv, bv, t2_ref, u_ref):
    agg = agg_ref[0] + agg_ref[1]
    den = den_ref[0, :, 0:16] + den_ref[1, :, 0:16]
    nf = nf_ref[...]
    denb = jnp.dot(den, den_m[...], precision=_HI,
                   preferred_element_type=jnp.float32)
    comb = agg / (denb + 1e-16) + jnp.dot(
        nf, ws[...], preferred_element_type=jnp.float32) + bs[...]
    mu = jnp.mean(comb, axis=-1, keepdims=True)
    cc = comb - mu
    var = jnp.mean(cc * cc, axis=-1, keepdims=True)
    comb = cc / jnp.sqrt(var + 1e-5) * g[...] + b[...]
    nf2 = comb + nf
    u_ref[:, 0:128] = comb
    u_ref[:, 128:256] = nf2
    t2_ref[:, 0:128] = jnp.dot(nf2, wq[...], preferred_element_type=jnp.float32) + bq[...]
    t2_ref[:, 128:256] = jnp.dot(nf2, wk[...], preferred_element_type=jnp.float32) + bk[...]
    t2_ref[:, 256:384] = jnp.dot(nf2, wv[...], preferred_element_type=jnp.float32) + bv[...]


def _finalize1(aggp, denp, nf, ptc1, ln, ptc2):
    args = (aggp, denp, nf,
            ptc1["skip"][0], ptc1["skip"][1].reshape(1, -1),
            ln[0].reshape(1, -1), ln[1].reshape(1, -1), jnp.asarray(_DEN),
            ptc2["q"][0], ptc2["q"][1].reshape(1, -1),
            ptc2["k"][0], ptc2["k"][1].reshape(1, -1),
            ptc2["v"][0], ptc2["v"][1].reshape(1, -1))
    return pl.pallas_call(
        _finalize1_body,
        grid=(_N // _NB,),
        in_specs=[pl.BlockSpec((2, _NB, 128), lambda i: (0, i, 0)),
                  pl.BlockSpec((2, _NB, 128), lambda i: (0, i, 0)),
                  _rows(_NB, 128)] + [_full(a) for a in args[3:]],
        out_specs=[_rows(_NB, 384), _rows(_NB, 256)],
        out_shape=[jax.ShapeDtypeStruct((_N, 384), jnp.float32),
                   jax.ShapeDtypeStruct((_N, 256), jnp.float32)],
        compiler_params=_cparams(),
    )(*args)


def _edge_stage_body(us_ref, ud_ref, ef0_ref,
                     wm1a, wm1b, bm1, wm2, bm2,
                     we1a, we1b, be1, we2, be2, wp, bp,
                     e2f_ref, e2b_ref, mom_ref):
    cs = us_ref[:, 0:128]
    cd = ud_ref[:, 0:128]
    ns = us_ref[:, 128:256]
    nd = ud_ref[:, 128:256]
    ef0 = ef0_ref[...]

    hm = jnp.maximum(jnp.dot(ns, wm1a[...], preferred_element_type=jnp.float32)
                     + jnp.dot(nd, wm1b[...], preferred_element_type=jnp.float32)
                     + bm1[...], 0.0)
    mom_ref[...] = jnp.dot(hm, wm2[...], preferred_element_type=jnp.float32) + bm2[...]

    def half(a, b, out_ref):
        h = jnp.maximum(jnp.dot(a, we1a[...], preferred_element_type=jnp.float32)
                        + jnp.dot(b, we1b[...], preferred_element_type=jnp.float32)
                        + be1[...], 0.0)
        ne = jnp.maximum(jnp.dot(h, we2[...], preferred_element_type=jnp.float32)
                         + be2[...], 0.0)
        ef2 = ne + ef0
        out_ref[...] = jnp.dot(ef2, wp[...], preferred_element_type=jnp.float32) + bp[...]

    half(cs, cd, e2f_ref)
    half(cd, cs, e2b_ref)


def _edge_stage(u_s, u_d, ef0, pmom, pedge, ptc2):
    (wm1, bm1), (wm2, bm2) = pmom
    (we1, be1), (we2, be2) = pedge
    wm2p = jnp.pad(wm2, ((0, 0), (0, 7)))
    bm2p = jnp.pad(bm2, (0, 7)).reshape(1, 8)
    args = (u_s, u_d, ef0,
            wm1[:128], wm1[128:], bm1.reshape(1, -1), wm2p, bm2p,
            we1[:128], we1[128:], be1.reshape(1, -1), we2,
            be2.reshape(1, -1),
            ptc2["e"][0], ptc2["e"][1].reshape(1, -1))
    return pl.pallas_call(
        _edge_stage_body,
        grid=(_E // _EB,),
        in_specs=[_rows(_EB, 256), _rows(_EB, 256), _rows(_EB, 128)]
        + [_full(a) for a in args[3:]],
        out_specs=[_rows(_EB, 128), _rows(_EB, 128), _rows(_EB, 8)],
        out_shape=[jax.ShapeDtypeStruct((_E, 128), jnp.float32),
                   jax.ShapeDtypeStruct((_E, 128), jnp.float32),
                   jax.ShapeDtypeStruct((_E, 8), jnp.float32)],
        compiler_params=_cparams(),
    )(*args)


def _finalize2_body(agg_ref, den_ref, u_ref, ws, bs, den_m, fin_ref):
    agg = agg_ref[0] + agg_ref[1]
    den = den_ref[0, :, 0:16] + den_ref[1, :, 0:16]
    nf2 = u_ref[:, 128:256]
    denb = jnp.dot(den, den_m[...], precision=_HI,
                   preferred_element_type=jnp.float32)
    fin_ref[...] = agg / (denb + 1e-16) + jnp.dot(
        nf2, ws[...], preferred_element_type=jnp.float32) + bs[...]


def _finalize2(aggp, denp, u, ptc2):
    args = (aggp, denp, u, ptc2["skip"][0], ptc2["skip"][1].reshape(1, -1),
            jnp.asarray(_DEN))
    return pl.pallas_call(
        _finalize2_body,
        grid=(_N // _NB,),
        in_specs=[pl.BlockSpec((2, _NB, 128), lambda i: (0, i, 0)),
                  pl.BlockSpec((2, _NB, 128), lambda i: (0, i, 0)),
                  _rows(_NB, 256)] + [_full(a) for a in args[3:]],
        out_specs=[_rows(_NB, 128)],
        out_shape=[jax.ShapeDtypeStruct((_N, 128), jnp.float32)],
        compiler_params=_cparams(),
    )(*args)[0]


def _head_body(fs_ref, fd_ref, w1a, w1b, b1, w2, b2, out_ref):
    h = jnp.maximum(jnp.dot(fs_ref[...], w1a[...], preferred_element_type=jnp.float32)
                    + jnp.dot(fd_ref[...], w1b[...], preferred_element_type=jnp.float32)
                    + b1[...], 0.0)
    out_ref[...] = jnp.dot(h, w2[...], preferred_element_type=jnp.float32) + b2[...]


def _head(f_s, f_d, p):
    (w1, b1), (w2, b2) = p
    w2p = jnp.pad(w2, ((0, 0), (0, 7)))
    b2p = jnp.pad(b2, (0, 7)).reshape(1, 8)
    args = (f_s, f_d, w1[:128], w1[128:], b1.reshape(1, -1), w2p, b2p)
    return pl.pallas_call(
        _head_body,
        grid=(_E // _EB,),
        in_specs=[_rows(_EB, 128), _rows(_EB, 128)]
        + [_full(a) for a in args[2:]],
        out_specs=[_rows(_EB, 8)],
        out_shape=[jax.ShapeDtypeStruct((_E, 8), jnp.float32)],
        compiler_params=_cparams(),
    )(*args)[0]


def _sc_gather(table, idx):
    if _SC_JNP:
        return jnp.take(table, idx, axis=0)
    m = idx.shape[0]
    r, d = table.shape
    nblk = d // 128
    per_w = m // _NW
    nch = per_w // _GCH
    r0 = (r // 16) // 8 * 8
    r_last = r - 15 * r0

    @functools.partial(
        pl.kernel,
        out_type=jax.ShapeDtypeStruct((m, d), jnp.float32),
        mesh=plsc.VectorSubcoreMesh(core_axis_name="c", subcore_axis_name="s"),
        scratch_types=[pltpu.VMEM((per_w,), jnp.int32),
                       pltpu.VMEM((_GCH, 128), jnp.float32),
                       pltpu.VMEM_SHARED((r, 128), jnp.float32),
                       pltpu.SemaphoreType.DMA],
    )
    def k(table_hbm, idx_hbm, out_hbm, idx_v, rows_v, tab_sh, sem):
        c = lax.axis_index("c")
        s = lax.axis_index("s")
        base = (s * 2 + c) * per_w
        pltpu.sync_copy(idx_hbm.at[pl.ds(base, per_w)], idx_v)
        for kb in range(nblk):
            cb = pl.ds(kb * 128, 128)

            @pl.when(s < 15)
            def _():
                pltpu.sync_copy(table_hbm.at[pl.ds(s * r0, r0), cb],
                                tab_sh.at[pl.ds(s * r0, r0)])

            @pl.when(s == 15)
            def _():
                pltpu.sync_copy(table_hbm.at[pl.ds(15 * r0, r_last), cb],
                                tab_sh.at[pl.ds(15 * r0, r_last)])

            plsc.subcore_barrier()

            @pl.loop(0, nch)
            def _(i):
                pltpu.async_copy(
                    tab_sh.at[idx_v.at[pl.ds(i * _GCH, _GCH)]], rows_v,
                    sem).wait()
                pltpu.sync_copy(
                    rows_v,
                    out_hbm.at[pl.ds(base + i * _GCH, _GCH), cb])

            plsc.subcore_barrier()

    return k(table, idx)


def _sc_scatter_add(msg_f, idx_f, msg_b, idx_b):
    if _SC_JNP or _SCATTER_JNP:
        seg = jnp.zeros((_NP, 128), jnp.float32).at[idx_f].add(msg_f).at[idx_b].add(msg_b)
        return jnp.stack([seg, jnp.zeros_like(seg)])
    m = msg_f.shape[0]
    half = m // 2
    per_s = half // 16
    nch = per_s // _GCH
    zr = _NP // 16
    nzch = zr // _GCH
    zeros = jnp.zeros((_GCH, 128), jnp.float32)

    @functools.partial(
        pl.kernel,
        out_type=jax.ShapeDtypeStruct((2, _NP, 128), jnp.float32),
        mesh=plsc.VectorSubcoreMesh(core_axis_name="c", subcore_axis_name="s"),
        scratch_types=[pltpu.VMEM((_GCH,), jnp.int32),
                       pltpu.VMEM((_GCH, 128), jnp.float32),
                       pltpu.VMEM_SHARED((_NP, 128), jnp.float32),
                       pltpu.SemaphoreType.DMA],
    )
    def k(mf_hbm, if_hbm, mb_hbm, ib_hbm, z_hbm,
          agg_hbm, idx_v, rows_v, acc_sh, sem):
        c = lax.axis_index("c")
        s = lax.axis_index("s")
        pltpu.sync_copy(z_hbm, rows_v)

        @pl.loop(0, nzch)
        def _(j):
            pltpu.sync_copy(rows_v, acc_sh.at[pl.ds(s * zr + j * _GCH, _GCH)])

        plsc.subcore_barrier()
        base = c * half + s * per_s
        for msg_hbm, i_hbm in ((mf_hbm, if_hbm), (mb_hbm, ib_hbm)):
            @pl.loop(0, nch)
            def _(i, msg_hbm=msg_hbm, i_hbm=i_hbm):
                b = base + i * _GCH
                pltpu.sync_copy(i_hbm.at[pl.ds(b, _GCH)], idx_v)
                pltpu.sync_copy(msg_hbm.at[pl.ds(b, _GCH)], rows_v)
                pltpu.sync_copy(rows_v, acc_sh.at[idx_v], add=True)
        plsc.subcore_barrier()

        @pl.loop(0, nzch)
        def _(j):
            o = s * zr + j * _GCH
            pltpu.sync_copy(acc_sh.at[pl.ds(o, _GCH)], rows_v)
            pltpu.sync_copy(rows_v, agg_hbm.at[c, pl.ds(o, _GCH)])

    return k(msg_f, idx_f, msg_b, idx_b, zeros)


def kernel(x, edge_index, edge_attr, params):
    src0 = edge_index[0]
    dst0 = edge_index[1]

    t1, nf = _node_prep1(x, params["node_emb"], params["mom_tc"])
    ef0, e1 = _edge_emb(edge_attr, params["edge_emb"], params["mom_tc"])

    g_s = _sc_gather(t1, src0)
    g_d = _sc_gather(t1, dst0)
    m_f, x_f, m_b, x_b = _attn(g_s, g_d, e1, e1)
    aggp = _sc_scatter_add(m_f, dst0, m_b, src0)
    denp = _sc_scatter_add(x_f, dst0, x_b, src0)
    t2, u = _finalize1(aggp, denp, nf, params["mom_tc"], params["mom_ln"],
                       params["edge_tc"])

    u_s = _sc_gather(u, src0)
    u_d = _sc_gather(u, dst0)
    e2_f, e2_b, mom8 = _edge_stage(u_s, u_d, ef0, params["mom_reg"],
                                   params["edge_mlp"], params["edge_tc"])

    t_s = _sc_gather(t2, src0)
    t_d = _sc_gather(t2, dst0)
    m_f2, x_f2, m_b2, x_b2 = _attn(t_s, t_d, e2_f, e2_b)
    aggp2 = _sc_scatter_add(m_f2, dst0, m_b2, src0)
    denp2 = _sc_scatter_add(x_f2, dst0, x_b2, src0)
    fin = _finalize2(aggp2, denp2, u, params["edge_tc"])

    f_s = _sc_gather(fin, src0)
    f_d = _sc_gather(fin, dst0)
    sc8 = _head(f_s, f_d, params["edge_cls"])

    return (mom8[:, 0], sc8[:, 0])


# speedup vs baseline: 27.0294x; 1.1183x over previous
import functools

import numpy as np
import jax
import jax.numpy as jnp
from jax import lax
from jax.experimental import pallas as pl
from jax.experimental.pallas import tpu as pltpu
from jax.experimental.pallas import tpu_sc as plsc

_N = 10000
_E = 320000
_H = 128
_HEADS = 4
_C = 32

_NB = 1000
_EB = 1280

_NP = 10240

_NW = 32
_GCH = 80
_NBUF = 5

_HI = jax.lax.Precision.HIGHEST

_SUMM = np.zeros((128, 4), np.float32)
for _h in range(4):
    _SUMM[32 * _h:32 * (_h + 1), _h] = 1.0
_BC = _SUMM.T.copy()
_EX128 = np.eye(4, 128, dtype=np.float32)
_DEN = np.zeros((16, 128), np.float32)
_DEN[:4] = _BC


def _full(a):
    nd = a.ndim
    return pl.BlockSpec(a.shape, lambda i, *_: (0,) * nd)


def _rows(block, ncols):
    return pl.BlockSpec((block, ncols), lambda i: (i, 0))


def _cparams():
    return pltpu.CompilerParams(dimension_semantics=("arbitrary",))


def _node_prep1_body(x_ref, w1, b1, w2, b2, wq, bq, wk, bk, wv, bv,
                     t1_ref, nf_ref):
    x = x_ref[...]
    h = jnp.maximum(jnp.dot(x, w1[...], preferred_element_type=jnp.float32)
                    + b1[...], 0.0)
    nf = jnp.maximum(jnp.dot(h, w2[...], preferred_element_type=jnp.float32)
                     + b2[...], 0.0)
    nf_ref[...] = nf
    t1_ref[:, 0:128] = jnp.dot(nf, wq[...], preferred_element_type=jnp.float32) + bq[...]
    t1_ref[:, 128:256] = jnp.dot(nf, wk[...], preferred_element_type=jnp.float32) + bk[...]
    t1_ref[:, 256:384] = jnp.dot(nf, wv[...], preferred_element_type=jnp.float32) + bv[...]


def _node_prep1(x, pn, ptc):
    (w1, b1), (w2, b2) = pn
    args = (x, w1, b1.reshape(1, -1), w2, b2.reshape(1, -1),
            ptc["q"][0], ptc["q"][1].reshape(1, -1),
            ptc["k"][0], ptc["k"][1].reshape(1, -1),
            ptc["v"][0], ptc["v"][1].reshape(1, -1))
    return pl.pallas_call(
        _node_prep1_body,
        grid=(_N // _NB,),
        in_specs=[_rows(_NB, 128)] + [_full(a) for a in args[1:]],
        out_specs=[_rows(_NB, 384), _rows(_NB, 128)],
        out_shape=[jax.ShapeDtypeStruct((_N, 384), jnp.float32),
                   jax.ShapeDtypeStruct((_N, 128), jnp.float32)],
        compiler_params=_cparams(),
    )(*args)


def _edge_emb_body(ea_ref, w1, b1, w2, b2, wp, bp, ef_ref, e1_ref):
    ea = ea_ref[...]
    h = jnp.maximum(jnp.dot(ea, w1[...], preferred_element_type=jnp.float32)
                    + b1[...], 0.0)
    ef = jnp.maximum(jnp.dot(h, w2[...], preferred_element_type=jnp.float32)
                     + b2[...], 0.0)
    ef_ref[...] = ef
    e1_ref[...] = jnp.dot(ef, wp[...], preferred_element_type=jnp.float32) + bp[...]


def _edge_emb(ea, pe, ptc):
    (w1, b1), (w2, b2) = pe
    args = (ea, w1, b1.reshape(1, -1), w2, b2.reshape(1, -1),
            ptc["e"][0], ptc["e"][1].reshape(1, -1))
    return pl.pallas_call(
        _edge_emb_body,
        grid=(_E // _EB,),
        in_specs=[_rows(_EB, 16)] + [_full(a) for a in args[1:]],
        out_specs=[_rows(_EB, 128), _rows(_EB, 128)],
        out_shape=[jax.ShapeDtypeStruct((_E, 128), jnp.float32),
                   jax.ShapeDtypeStruct((_E, 128), jnp.float32)],
        compiler_params=_cparams(),
    )(*args)


def _attn_body(gs_ref, gd_ref, ef_ref, eb_ref, summ, bc, ex128,
               mf_ref, xf_ref, mb_ref, xb_ref):
    isq = 1.0 / np.sqrt(np.float32(_C))
    gs = gs_ref[...]
    gd = gd_ref[...]

    def half(q, k, v, e, m_ref, x_ref):
        prod = q * (k + e)
        alpha = jnp.dot(prod, summ[...], precision=_HI,
                        preferred_element_type=jnp.float32) * isq
        ex = jnp.exp(alpha)
        exb = jnp.dot(ex, bc[...], precision=_HI,
                      preferred_element_type=jnp.float32)
        m_ref[...] = (v + e) * exb
        x_ref[...] = jnp.dot(ex, ex128[...], precision=_HI,
                             preferred_element_type=jnp.float32)

    half(gd[:, 0:128], gs[:, 128:256], gs[:, 256:384], ef_ref[...],
         mf_ref, xf_ref)
    half(gs[:, 0:128], gd[:, 128:256], gd[:, 256:384], eb_ref[...],
         mb_ref, xb_ref)


def _attn(gs, gd, e_f, e_b):
    args = (gs, gd, e_f, e_b, jnp.asarray(_SUMM), jnp.asarray(_BC),
            jnp.asarray(_EX128))
    return pl.pallas_call(
        _attn_body,
        grid=(_E // _EB,),
        in_specs=[_rows(_EB, 384), _rows(_EB, 384), _rows(_EB, 128),
                  _rows(_EB, 128)] + [_full(a) for a in args[4:]],
        out_specs=[_rows(_EB, 128), _rows(_EB, 128),
                   _rows(_EB, 128), _rows(_EB, 128)],
        out_shape=[jax.ShapeDtypeStruct((_E, 128), jnp.float32),
                   jax.ShapeDtypeStruct((_E, 128), jnp.float32),
                   jax.ShapeDtypeStruct((_E, 128), jnp.float32),
                   jax.ShapeDtypeStruct((_E, 128), jnp.float32)],
        compiler_params=_cparams(),
    )(*args)


def _finalize1_body(agg_ref, den_ref, nf_ref, ws, bs, g, b, den_m,
                    wq, bq, wk, bk, wv, bv, t2_ref, u_ref):
    agg = agg_ref[0] + agg_ref[1]
    den = den_ref[0, :, 0:16] + den_ref[1, :, 0:16]
    nf = nf_ref[...]
    denb = jnp.dot(den, den_m[...], precision=_HI,
                   preferred_element_type=jnp.float32)
    comb = agg / (denb + 1e-16) + jnp.dot(
        nf, ws[...], preferred_element_type=jnp.float32) + bs[...]
    mu = jnp.mean(comb, axis=-1, keepdims=True)
    cc = comb - mu
    var = jnp.mean(cc * cc, axis=-1, keepdims=True)
    comb = cc / jnp.sqrt(var + 1e-5) * g[...] + b[...]
    nf2 = comb + nf
    u_ref[:, 0:128] = comb
    u_ref[:, 128:256] = nf2
    t2_ref[:, 0:128] = jnp.dot(nf2, wq[...], preferred_element_type=jnp.float32) + bq[...]
    t2_ref[:, 128:256] = jnp.dot(nf2, wk[...], preferred_element_type=jnp.float32) + bk[...]
    t2_ref[:, 256:384] = jnp.dot(nf2, wv[...], preferred_element_type=jnp.float32) + bv[...]


def _finalize1(aggp, denp, nf, ptc1, ln, ptc2):
    args = (aggp, denp, nf,
            ptc1["skip"][0], ptc1["skip"][1].reshape(1, -1),
            ln[0].reshape(1, -1), ln[1].reshape(1, -1), jnp.asarray(_DEN),
            ptc2["q"][0], ptc2["q"][1].reshape(1, -1),
            ptc2["k"][0], ptc2["k"][1].reshape(1, -1),
            ptc2["v"][0], ptc2["v"][1].reshape(1, -1))
    return pl.pallas_call(
        _finalize1_body,
        grid=(_N // _NB,),
        in_specs=[pl.BlockSpec((2, _NB, 128), lambda i: (0, i, 0)),
                  pl.BlockSpec((2, _NB, 128), lambda i: (0, i, 0)),
                  _rows(_NB, 128)] + [_full(a) for a in args[3:]],
        out_specs=[_rows(_NB, 384), _rows(_NB, 256)],
        out_shape=[jax.ShapeDtypeStruct((_N, 384), jnp.float32),
                   jax.ShapeDtypeStruct((_N, 256), jnp.float32)],
        compiler_params=_cparams(),
    )(*args)


def _edge_stage_body(us_ref, ud_ref, ef0_ref,
                     wm1a, wm1b, bm1, wm2, bm2,
                     we1a, we1b, be1, we2, be2, wp, bp,
                     e2f_ref, e2b_ref, mom_ref):
    cs = us_ref[:, 0:128]
    cd = ud_ref[:, 0:128]
    ns = us_ref[:, 128:256]
    nd = ud_ref[:, 128:256]
    ef0 = ef0_ref[...]

    hm = jnp.maximum(jnp.dot(ns, wm1a[...], preferred_element_type=jnp.float32)
                     + jnp.dot(nd, wm1b[...], preferred_element_type=jnp.float32)
                     + bm1[...], 0.0)
    mom_ref[...] = jnp.dot(hm, wm2[...], preferred_element_type=jnp.float32) + bm2[...]

    def half(a, b, out_ref):
        h = jnp.maximum(jnp.dot(a, we1a[...], preferred_element_type=jnp.float32)
                        + jnp.dot(b, we1b[...], preferred_element_type=jnp.float32)
                        + be1[...], 0.0)
        ne = jnp.maximum(jnp.dot(h, we2[...], preferred_element_type=jnp.float32)
                         + be2[...], 0.0)
        ef2 = ne + ef0
        out_ref[...] = jnp.dot(ef2, wp[...], preferred_element_type=jnp.float32) + bp[...]

    half(cs, cd, e2f_ref)
    half(cd, cs, e2b_ref)


def _edge_stage(u_s, u_d, ef0, pmom, pedge, ptc2):
    (wm1, bm1), (wm2, bm2) = pmom
    (we1, be1), (we2, be2) = pedge
    wm2p = jnp.pad(wm2, ((0, 0), (0, 7)))
    bm2p = jnp.pad(bm2, (0, 7)).reshape(1, 8)
    args = (u_s, u_d, ef0,
            wm1[:128], wm1[128:], bm1.reshape(1, -1), wm2p, bm2p,
            we1[:128], we1[128:], be1.reshape(1, -1), we2,
            be2.reshape(1, -1),
            ptc2["e"][0], ptc2["e"][1].reshape(1, -1))
    return pl.pallas_call(
        _edge_stage_body,
        grid=(_E // _EB,),
        in_specs=[_rows(_EB, 256), _rows(_EB, 256), _rows(_EB, 128)]
        + [_full(a) for a in args[3:]],
        out_specs=[_rows(_EB, 128), _rows(_EB, 128), _rows(_EB, 8)],
        out_shape=[jax.ShapeDtypeStruct((_E, 128), jnp.float32),
                   jax.ShapeDtypeStruct((_E, 128), jnp.float32),
                   jax.ShapeDtypeStruct((_E, 8), jnp.float32)],
        compiler_params=_cparams(),
    )(*args)


def _finalize2_body(agg_ref, den_ref, u_ref, ws, bs, den_m, fin_ref):
    agg = agg_ref[0] + agg_ref[1]
    den = den_ref[0, :, 0:16] + den_ref[1, :, 0:16]
    nf2 = u_ref[:, 128:256]
    denb = jnp.dot(den, den_m[...], precision=_HI,
                   preferred_element_type=jnp.float32)
    fin_ref[...] = agg / (denb + 1e-16) + jnp.dot(
        nf2, ws[...], preferred_element_type=jnp.float32) + bs[...]


def _finalize2(aggp, denp, u, ptc2):
    args = (aggp, denp, u, ptc2["skip"][0], ptc2["skip"][1].reshape(1, -1),
            jnp.asarray(_DEN))
    return pl.pallas_call(
        _finalize2_body,
        grid=(_N // _NB,),
        in_specs=[pl.BlockSpec((2, _NB, 128), lambda i: (0, i, 0)),
                  pl.BlockSpec((2, _NB, 128), lambda i: (0, i, 0)),
                  _rows(_NB, 256)] + [_full(a) for a in args[3:]],
        out_specs=[_rows(_NB, 128)],
        out_shape=[jax.ShapeDtypeStruct((_N, 128), jnp.float32)],
        compiler_params=_cparams(),
    )(*args)[0]


def _head_body(fs_ref, fd_ref, w1a, w1b, b1, w2, b2, out_ref):
    h = jnp.maximum(jnp.dot(fs_ref[...], w1a[...], preferred_element_type=jnp.float32)
                    + jnp.dot(fd_ref[...], w1b[...], preferred_element_type=jnp.float32)
                    + b1[...], 0.0)
    out_ref[...] = jnp.dot(h, w2[...], preferred_element_type=jnp.float32) + b2[...]


def _head(f_s, f_d, p):
    (w1, b1), (w2, b2) = p
    w2p = jnp.pad(w2, ((0, 0), (0, 7)))
    b2p = jnp.pad(b2, (0, 7)).reshape(1, 8)
    args = (f_s, f_d, w1[:128], w1[128:], b1.reshape(1, -1), w2p, b2p)
    return pl.pallas_call(
        _head_body,
        grid=(_E // _EB,),
        in_specs=[_rows(_EB, 128), _rows(_EB, 128)]
        + [_full(a) for a in args[2:]],
        out_specs=[_rows(_EB, 8)],
        out_shape=[jax.ShapeDtypeStruct((_E, 8), jnp.float32)],
        compiler_params=_cparams(),
    )(*args)[0]


def _sc_gather(table, idx):
    m = idx.shape[0]
    d = table.shape[1]
    per_w = m // _NW
    gch = 40 if d > 256 else _GCH
    nch = per_w // gch

    @functools.partial(
        pl.kernel,
        out_type=jax.ShapeDtypeStruct((m, d), jnp.float32),
        mesh=plsc.VectorSubcoreMesh(core_axis_name="c", subcore_axis_name="s"),
        scratch_types=[pltpu.VMEM((per_w,), jnp.int32),
                       pltpu.VMEM((_NBUF, gch, d), jnp.float32),
                       pltpu.SemaphoreType.DMA,
                       pltpu.SemaphoreType.DMA],
    )
    def k(table_hbm, idx_hbm, out_hbm, idx_v, buf_v, gsem, wsem):
        c = lax.axis_index("c")
        s = lax.axis_index("s")
        base = (s * 2 + c) * per_w
        pltpu.sync_copy(idx_hbm.at[pl.ds(base, per_w)], idx_v)

        @pl.loop(0, nch, step=_NBUF)
        def _(i):
            gs = [pltpu.async_copy(
                table_hbm.at[idx_v.at[pl.ds((i + b) * gch, gch)]],
                buf_v.at[b], gsem) for b in range(_NBUF)]
            for dsc in gs:
                dsc.wait()
            ws = [pltpu.async_copy(
                buf_v.at[b],
                out_hbm.at[pl.ds(base + (i + b) * gch, gch)],
                wsem) for b in range(_NBUF)]
            for dsc in ws:
                dsc.wait()

    return k(table, idx)


def _sc_scatter_add(msg_f, idx_f, msg_b, idx_b):
    m = msg_f.shape[0]
    half = m // 2
    per_s = half // 16
    nch = per_s // _GCH
    zr = _NP // 16
    nzch = zr // _GCH
    zeros = jnp.zeros((_GCH, 128), jnp.float32)

    @functools.partial(
        pl.kernel,
        out_type=jax.ShapeDtypeStruct((2, _NP, 128), jnp.float32),
        mesh=plsc.VectorSubcoreMesh(core_axis_name="c", subcore_axis_name="s"),
        scratch_types=[pltpu.VMEM((_GCH,), jnp.int32),
                       pltpu.VMEM((_GCH, 128), jnp.float32),
                       pltpu.VMEM_SHARED((_NP, 128), jnp.float32),
                       pltpu.SemaphoreType.DMA],
    )
    def k(mf_hbm, if_hbm, mb_hbm, ib_hbm, z_hbm,
          agg_hbm, idx_v, rows_v, acc_sh, sem):
        c = lax.axis_index("c")
        s = lax.axis_index("s")
        pltpu.sync_copy(z_hbm, rows_v)

        @pl.loop(0, nzch)
        def _(j):
            pltpu.sync_copy(rows_v, acc_sh.at[pl.ds(s * zr + j * _GCH, _GCH)])

        plsc.subcore_barrier()
        base = c * half + s * per_s
        for msg_hbm, i_hbm in ((mf_hbm, if_hbm), (mb_hbm, ib_hbm)):
            @pl.loop(0, nch)
            def _(i, msg_hbm=msg_hbm, i_hbm=i_hbm):
                b = base + i * _GCH
                pltpu.sync_copy(i_hbm.at[pl.ds(b, _GCH)], idx_v)
                pltpu.sync_copy(msg_hbm.at[pl.ds(b, _GCH)], rows_v)
                pltpu.sync_copy(rows_v, acc_sh.at[idx_v], add=True)
        plsc.subcore_barrier()

        @pl.loop(0, nzch)
        def _(j):
            o = s * zr + j * _GCH
            pltpu.sync_copy(acc_sh.at[pl.ds(o, _GCH)], rows_v)
            pltpu.sync_copy(rows_v, agg_hbm.at[c, pl.ds(o, _GCH)])

    return k(msg_f, idx_f, msg_b, idx_b, zeros)


def kernel(x, edge_index, edge_attr, params):
    src0 = edge_index[0]
    dst0 = edge_index[1]

    t1, nf = _node_prep1(x, params["node_emb"], params["mom_tc"])
    ef0, e1 = _edge_emb(edge_attr, params["edge_emb"], params["mom_tc"])

    g_s = _sc_gather(t1, src0)
    g_d = _sc_gather(t1, dst0)
    m_f, x_f, m_b, x_b = _attn(g_s, g_d, e1, e1)
    aggp = _sc_scatter_add(m_f, dst0, m_b, src0)
    denp = _sc_scatter_add(x_f, dst0, x_b, src0)
    t2, u = _finalize1(aggp, denp, nf, params["mom_tc"], params["mom_ln"],
                       params["edge_tc"])

    u_s = _sc_gather(u, src0)
    u_d = _sc_gather(u, dst0)
    e2_f, e2_b, mom8 = _edge_stage(u_s, u_d, ef0, params["mom_reg"],
                                   params["edge_mlp"], params["edge_tc"])

    t_s = _sc_gather(t2, src0)
    t_d = _sc_gather(t2, dst0)
    m_f2, x_f2, m_b2, x_b2 = _attn(t_s, t_d, e2_f, e2_b)
    aggp2 = _sc_scatter_add(m_f2, dst0, m_b2, src0)
    denp2 = _sc_scatter_add(x_f2, dst0, x_b2, src0)
    fin = _finalize2(aggp2, denp2, u, params["edge_tc"])

    f_s = _sc_gather(fin, src0)
    f_d = _sc_gather(fin, dst0)
    sc8 = _head(f_s, f_d, params["edge_cls"])

    return (mom8[:, 0], sc8[:, 0])
